# Initial kernel scaffold; baseline (speedup 1.0000x reference)
import jax
import jax.numpy as jnp
from jax.experimental import pallas as pl


def kernel(x, edge_index_rsr, edge_index_rtr, edge_index_rur, W1_rsr, b1_rsr, W1_rtr, b1_rtr, W1_rur, b1_rur, W2_rsr, b2_rsr, W2_rtr, b2_rtr, W2_rur, b2_rur):
    raise NotImplementedError("write your pallas kernel here")


# speedup vs baseline: 29.5010x; 29.5010x over previous
import functools

import jax
import jax.numpy as jnp
from jax import lax
from jax.experimental import pallas as pl
from jax.experimental.pallas import tpu as pltpu
from jax.experimental.pallas import tpu_sc as plsc

N = 10000
E = 320000
IN_F = 128
HID = 16
NPAD = 10240
NC = 2
NS = 16
NW = NC * NS
EW = E // NW
CH = 2000
NCH = EW // CH
SL = NPAD // NS

_mesh = plsc.VectorSubcoreMesh(
    core_axis_name="c", subcore_axis_name="s", num_cores=NC, num_subcores=NS)

_f32 = jnp.float32
_SC_PARAMS = pltpu.CompilerParams(use_tc_tiling_on_sc=False)


@functools.partial(
    pl.kernel,
    out_type=jax.ShapeDtypeStruct((NC, 6, NPAD), _f32),
    mesh=_mesh,
    compiler_params=_SC_PARAMS,
    scratch_types=(
        [pltpu.VMEM((EW,), jnp.int32), pltpu.VMEM((EW,), _f32)]
        + [pltpu.VMEM_SHARED((NPAD,), _f32) for _ in range(6)]
    ),
)
def _sc_degrees(e0, e1, e2, zn, ones_hbm, out, idx_v, ones_v, *hists):
    cid = lax.axis_index("c")
    sid = lax.axis_index("s")
    wid = sid * NC + cid
    for h in hists:
        pltpu.sync_copy(zn.at[pl.ds(sid * SL, SL)], h.at[pl.ds(sid * SL, SL)])
    pltpu.sync_copy(ones_hbm, ones_v)
    plsc.subcore_barrier()
    for t, ei in enumerate((e0, e1, e2)):
        for r in range(2):
            pltpu.sync_copy(ei.at[pl.ds(r * E + wid * EW, EW)], idx_v)
            pltpu.sync_copy(ones_v, hists[2 * t + r].at[idx_v], add=True)
    plsc.subcore_barrier()
    for hi, h in enumerate(hists):
        pltpu.sync_copy(h.at[pl.ds(sid * SL, SL)],
                        out.at[cid, hi, pl.ds(sid * SL, SL)])


@functools.partial(
    pl.kernel,
    out_type=jax.ShapeDtypeStruct((NC, 3, NPAD, HID), _f32),
    mesh=_mesh,
    compiler_params=_SC_PARAMS,
    scratch_types=(
        [pltpu.VMEM((CH,), jnp.int32), pltpu.VMEM((CH,), jnp.int32),
         pltpu.VMEM((CH, HID), _f32), pltpu.SemaphoreType.DMA]
        + [pltpu.VMEM_SHARED((NPAD, HID), _f32) for _ in range(3)]
    ),
)
def _sc_agg1(e0, e1, e2, y0, y1, y2, znk, out, sidx, didx, rows, sem, *accs):
    cid = lax.axis_index("c")
    sid = lax.axis_index("s")
    wid = sid * NC + cid
    for a in accs:
        pltpu.sync_copy(znk.at[pl.ds(sid * SL, SL)], a.at[pl.ds(sid * SL, SL)])
    plsc.subcore_barrier()
    for ei, y, a in zip((e0, e1, e2), (y0, y1, y2), accs):
        for k in range(NCH):
            base = wid * EW + k * CH
            pltpu.sync_copy(ei.at[pl.ds(base, CH)], sidx)
            pltpu.sync_copy(ei.at[pl.ds(E + base, CH)], didx)
            pltpu.async_copy(y.at[sidx], rows, sem).wait()
            pltpu.sync_copy(rows, a.at[didx], add=True)
    plsc.subcore_barrier()
    for t, a in enumerate(accs):
        pltpu.sync_copy(a.at[pl.ds(sid * SL, SL)],
                        out.at[cid, t, pl.ds(sid * SL, SL)])


@functools.partial(
    pl.kernel,
    out_type=jax.ShapeDtypeStruct((NC, 3, NPAD), _f32),
    mesh=_mesh,
    compiler_params=_SC_PARAMS,
    scratch_types=(
        [pltpu.VMEM((CH,), jnp.int32), pltpu.VMEM((CH,), jnp.int32),
         pltpu.VMEM((CH,), _f32), pltpu.SemaphoreType.DMA]
        + [pltpu.VMEM_SHARED((NPAD,), _f32) for _ in range(3)]
    ),
)
def _sc_agg2(e0, e1, e2, z0, z1, z2, zn, out, sidx, didx, vals, sem, *accs):
    cid = lax.axis_index("c")
    sid = lax.axis_index("s")
    wid = sid * NC + cid
    for a in accs:
        pltpu.sync_copy(zn.at[pl.ds(sid * SL, SL)], a.at[pl.ds(sid * SL, SL)])
    plsc.subcore_barrier()
    for ei, z, a in zip((e0, e1, e2), (z0, z1, z2), accs):
        for k in range(NCH):
            base = wid * EW + k * CH
            pltpu.sync_copy(ei.at[pl.ds(base, CH)], sidx)
            pltpu.sync_copy(ei.at[pl.ds(E + base, CH)], didx)
            pltpu.async_copy(z.at[sidx], vals, sem).wait()
            pltpu.sync_copy(vals, a.at[didx], add=True)
    plsc.subcore_barrier()
    for t, a in enumerate(accs):
        pltpu.sync_copy(a.at[pl.ds(sid * SL, SL)],
                        out.at[cid, t, pl.ds(sid * SL, SL)])


def _tc1_body(degp_ref, x_ref, w_ref, dinv_ref, y0_ref, y1_ref, y2_ref):
    deg = jnp.maximum(degp_ref[0] + degp_ref[1], 1.0)
    dinv = lax.rsqrt(deg)
    dinv_ref[...] = dinv
    y = jnp.dot(x_ref[...], w_ref[...], preferred_element_type=_f32)
    for e, yr in enumerate((y0_ref, y1_ref, y2_ref)):
        yr[...] = y[:, 16 * e:16 * (e + 1)] * dinv[2 * e][:, None]


def _tc2_body(aggp_ref, dinv_ref, b1_ref, w2_ref, z0_ref, z1_ref, z2_ref):
    dinv = dinv_ref[...]
    h = jnp.zeros((NPAD, HID), _f32)
    for e in range(3):
        h = h + (aggp_ref[0, e] + aggp_ref[1, e]) * dinv[2 * e + 1][:, None]
    h = jnp.maximum(h + jnp.sum(b1_ref[...], axis=0)[None, :], 0.0)
    for e, zr in enumerate((z0_ref, z1_ref, z2_ref)):
        zr[...] = jnp.dot(h, w2_ref[:, e:e + 1],
                          preferred_element_type=_f32) * dinv[2 * e][:, None]


def _tc3_body(oaccp_ref, dinv_ref, b2_ref, out_ref):
    dinv = dinv_ref[...]
    o = jnp.zeros((NPAD,), _f32)
    for e in range(3):
        o = o + (oaccp_ref[0, e] + oaccp_ref[1, e]) * dinv[2 * e + 1]
    out_ref[...] = o + b2_ref[0, 0]


def kernel(x, edge_index_rsr, edge_index_rtr, edge_index_rur,
           W1_rsr, b1_rsr, W1_rtr, b1_rtr, W1_rur, b1_rur,
           W2_rsr, b2_rsr, W2_rtr, b2_rtr, W2_rur, b2_rur):
    xpad = jnp.zeros((NPAD, IN_F), _f32).at[:N].set(x)
    w1 = jnp.concatenate([W1_rsr, W1_rtr, W1_rur], axis=1)
    w2 = jnp.concatenate([W2_rsr, W2_rtr, W2_rur], axis=1)
    b1 = jnp.stack([b1_rsr, b1_rtr, b1_rur])
    b2 = (b2_rsr + b2_rtr + b2_rur).reshape(1, 1)
    zn = jnp.zeros((NPAD,), _f32)
    znk = jnp.zeros((NPAD, HID), _f32)
    ones = jnp.ones((EW,), _f32)
    es = (edge_index_rsr.reshape(2 * E), edge_index_rtr.reshape(2 * E),
          edge_index_rur.reshape(2 * E))

    degp = _sc_degrees(*es, zn, ones)

    dinv, y0, y1, y2 = pl.pallas_call(
        _tc1_body,
        out_shape=(jax.ShapeDtypeStruct((6, NPAD), _f32),
                   jax.ShapeDtypeStruct((NPAD, HID), _f32),
                   jax.ShapeDtypeStruct((NPAD, HID), _f32),
                   jax.ShapeDtypeStruct((NPAD, HID), _f32)),
    )(degp, xpad, w1)

    aggp = _sc_agg1(*es, y0, y1, y2, znk)

    z0, z1, z2 = pl.pallas_call(
        _tc2_body,
        out_shape=(jax.ShapeDtypeStruct((NPAD, 1), _f32),
                   jax.ShapeDtypeStruct((NPAD, 1), _f32),
                   jax.ShapeDtypeStruct((NPAD, 1), _f32)),
    )(aggp, dinv, b1, w2)

    oaccp = _sc_agg2(*es, z0.reshape(NPAD), z1.reshape(NPAD),
                     z2.reshape(NPAD), zn)

    out = pl.pallas_call(
        _tc3_body,
        out_shape=jax.ShapeDtypeStruct((NPAD,), _f32),
    )(oaccp, dinv, b2)
    return out[:N]


# speedup vs baseline: 42.2128x; 1.4309x over previous
import functools

import jax
import jax.numpy as jnp
from jax import lax
from jax.experimental import pallas as pl
from jax.experimental.pallas import tpu as pltpu
from jax.experimental.pallas import tpu_sc as plsc

N = 10000
E = 320000
IN_F = 128
HID = 16
NPAD = 10240
NC = 2
NS = 16
NW = NC * NS
EW = E // NW
CH = 2000
NCH = EW // CH
SL = NPAD // NS

_mesh = plsc.VectorSubcoreMesh(
    core_axis_name="c", subcore_axis_name="s", num_cores=NC, num_subcores=NS)

_f32 = jnp.float32
_SC_PARAMS = pltpu.CompilerParams(use_tc_tiling_on_sc=False,
                                  needs_layout_passes=False)


@functools.partial(
    pl.kernel,
    out_type=jax.ShapeDtypeStruct((NC, 6, NPAD), _f32),
    mesh=_mesh,
    compiler_params=_SC_PARAMS,
    scratch_types=(
        [pltpu.VMEM((EW,), jnp.int32), pltpu.VMEM((EW,), _f32)]
        + [pltpu.VMEM_SHARED((NPAD,), _f32) for _ in range(6)]
    ),
)
def _sc_degrees(e0, e1, e2, zn, ones_hbm, out, idx_v, ones_v, *hists):
    cid = lax.axis_index("c")
    sid = lax.axis_index("s")
    wid = sid * NC + cid
    for h in hists:
        pltpu.sync_copy(zn.at[pl.ds(sid * SL, SL)], h.at[pl.ds(sid * SL, SL)])
    pltpu.sync_copy(ones_hbm, ones_v)
    plsc.subcore_barrier()
    for t, ei in enumerate((e0, e1, e2)):
        for r in range(2):
            pltpu.sync_copy(ei.at[pl.ds(r * E + wid * EW, EW)], idx_v)
            pltpu.sync_copy(ones_v, hists[2 * t + r].at[idx_v], add=True)
    plsc.subcore_barrier()
    for hi, h in enumerate(hists):
        pltpu.sync_copy(h.at[pl.ds(sid * SL, SL)],
                        out.at[cid, hi, pl.ds(sid * SL, SL)])


@functools.partial(
    pl.kernel,
    out_type=jax.ShapeDtypeStruct((NC, 3, NPAD, HID), _f32),
    mesh=_mesh,
    compiler_params=_SC_PARAMS,
    scratch_types=(
        [pltpu.VMEM((CH,), jnp.int32) for _ in range(4)]
        + [pltpu.VMEM((CH, HID), _f32) for _ in range(2)]
        + [pltpu.SemaphoreType.DMA for _ in range(4)]
        + [pltpu.VMEM_SHARED((NPAD, HID), _f32) for _ in range(3)]
    ),
)
def _sc_agg1(e0, e1, e2, y0, y1, y2, znk, out,
             sidx0, sidx1, didx0, didx1, rows0, rows1,
             gsem0, gsem1, ssem0, ssem1, *accs):
    cid = lax.axis_index("c")
    sid = lax.axis_index("s")
    wid = sid * NC + cid
    sidx = (sidx0, sidx1)
    didx = (didx0, didx1)
    rows = (rows0, rows1)
    gsem = (gsem0, gsem1)
    ssem = (ssem0, ssem1)
    for a in accs:
        pltpu.sync_copy(znk.at[pl.ds(sid * SL, SL)], a.at[pl.ds(sid * SL, SL)])
    plsc.subcore_barrier()
    ys = (y0, y1, y2)
    eis = (e0, e1, e2)
    chunks = [(t, k) for t in range(3) for k in range(NCH)]
    tot = len(chunks)

    def load_idx(cnt, b):
        t, k = chunks[cnt]
        base = wid * EW + k * CH
        pltpu.sync_copy(eis[t].at[pl.ds(base, CH)], sidx[b])
        pltpu.sync_copy(eis[t].at[pl.ds(E + base, CH)], didx[b])

    descs_g = [None] * tot
    load_idx(0, 0)
    descs_g[0] = pltpu.async_copy(ys[chunks[0][0]].at[sidx[0]], rows[0], gsem[0])
    for cnt in range(tot):
        b = cnt % 2
        nb = 1 - b
        if cnt + 1 < tot:
            load_idx(cnt + 1, nb)
            descs_g[cnt + 1] = pltpu.async_copy(
                ys[chunks[cnt + 1][0]].at[sidx[nb]], rows[nb], gsem[nb])
        descs_g[cnt].wait()
        pltpu.sync_copy(rows[b], accs[chunks[cnt][0]].at[didx[b]], add=True)
    plsc.subcore_barrier()
    for t, a in enumerate(accs):
        pltpu.sync_copy(a.at[pl.ds(sid * SL, SL)],
                        out.at[cid, t, pl.ds(sid * SL, SL)])


@functools.partial(
    pl.kernel,
    out_type=jax.ShapeDtypeStruct((NC, NPAD), _f32),
    mesh=_mesh,
    compiler_params=_SC_PARAMS,
    scratch_types=(
        [pltpu.VMEM((NPAD,), _f32) for _ in range(6)]
        + [pltpu.VMEM((CH,), jnp.int32) for _ in range(3)]
        + [pltpu.VMEM((CH,), _f32) for _ in range(2)]
        + [pltpu.SemaphoreType.DMA for _ in range(2)]
        + [pltpu.VMEM_SHARED((NPAD,), _f32)]
    ),
)
def _sc_agg2(e0, e1, e2, z0, z1, z2, di0, di1, di2, zn, out,
             zv0, zv1, zv2, dv0, dv1, dv2, sidx, didx0, didx1,
             vals0, vals1, ssem0, ssem1, acc):
    cid = lax.axis_index("c")
    sid = lax.axis_index("s")
    wid = sid * NC + cid
    didx = (didx0, didx1)
    vals = (vals0, vals1)
    ssem = (ssem0, ssem1)
    pltpu.sync_copy(zn.at[pl.ds(sid * SL, SL)], acc.at[pl.ds(sid * SL, SL)])
    for hbm, v in ((z0, zv0), (z1, zv1), (z2, zv2),
                   (di0, dv0), (di1, dv1), (di2, dv2)):
        pltpu.sync_copy(hbm, v)
    plsc.subcore_barrier()
    zvs = (zv0, zv1, zv2)
    dvs = (dv0, dv1, dv2)
    eis = (e0, e1, e2)
    chunks = [(t, k) for t in range(3) for k in range(NCH)]
    tot = len(chunks)
    descs_s = [None] * tot
    for cnt in range(tot):
        t, k = chunks[cnt]
        b = cnt % 2
        base = wid * EW + k * CH
        pltpu.sync_copy(eis[t].at[pl.ds(base, CH)], sidx)
        pltpu.sync_copy(eis[t].at[pl.ds(E + base, CH)], didx[b])
        zv = zvs[t]
        dv = dvs[t]
        vb = vals[b]
        db = didx[b]

        def body(i, _, zv=zv, dv=dv, vb=vb, db=db):
            idxs = sidx[pl.ds(i * 16, 16)]
            idxd = db[pl.ds(i * 16, 16)]
            v = plsc.load_gather(zv, [idxs]) * plsc.load_gather(dv, [idxd])
            vb[pl.ds(i * 16, 16)] = v
            return 0

        lax.fori_loop(0, CH // 16, body, 0)
        pltpu.sync_copy(vb, acc.at[db], add=True)
    plsc.subcore_barrier()
    pltpu.sync_copy(acc.at[pl.ds(sid * SL, SL)],
                    out.at[cid, pl.ds(sid * SL, SL)])


def _tc1_body(degp_ref, x_ref, w_ref, dinv_ref, y0_ref, y1_ref, y2_ref,
              di0_ref, di1_ref, di2_ref):
    deg = jnp.maximum(degp_ref[0] + degp_ref[1], 1.0)
    dinv = lax.rsqrt(deg)
    dinv_ref[...] = dinv
    y = jnp.dot(x_ref[...], w_ref[...], preferred_element_type=_f32)
    for e, yr in enumerate((y0_ref, y1_ref, y2_ref)):
        yr[...] = y[:, 16 * e:16 * (e + 1)] * dinv[2 * e][:, None]
    for e, dr in enumerate((di0_ref, di1_ref, di2_ref)):
        dr[...] = dinv[2 * e + 1]


def _tc2_body(aggp_ref, dinv_ref, b1_ref, w2_ref, z0_ref, z1_ref, z2_ref):
    dinv = dinv_ref[...]
    h = jnp.zeros((NPAD, HID), _f32)
    for e in range(3):
        h = h + (aggp_ref[0, e] + aggp_ref[1, e]) * dinv[2 * e + 1][:, None]
    h = jnp.maximum(h + jnp.sum(b1_ref[...], axis=0)[None, :], 0.0)
    for e, zr in enumerate((z0_ref, z1_ref, z2_ref)):
        zr[...] = jnp.dot(h, w2_ref[:, e:e + 1],
                          preferred_element_type=_f32) * dinv[2 * e][:, None]


def kernel(x, edge_index_rsr, edge_index_rtr, edge_index_rur,
           W1_rsr, b1_rsr, W1_rtr, b1_rtr, W1_rur, b1_rur,
           W2_rsr, b2_rsr, W2_rtr, b2_rtr, W2_rur, b2_rur):
    xpad = jnp.zeros((NPAD, IN_F), _f32).at[:N].set(x)
    w1 = jnp.concatenate([W1_rsr, W1_rtr, W1_rur], axis=1)
    w2 = jnp.concatenate([W2_rsr, W2_rtr, W2_rur], axis=1)
    b1 = jnp.stack([b1_rsr, b1_rtr, b1_rur])
    b2 = (b2_rsr + b2_rtr + b2_rur).reshape(1, 1)
    zn = jnp.zeros((NPAD,), _f32)
    znk = jnp.zeros((NPAD, HID), _f32)
    ones = jnp.ones((EW,), _f32)
    es = (edge_index_rsr.reshape(2 * E), edge_index_rtr.reshape(2 * E),
          edge_index_rur.reshape(2 * E))

    degp = _sc_degrees(*es, zn, ones)

    dinv, y0, y1, y2, di0, di1, di2 = pl.pallas_call(
        _tc1_body,
        out_shape=(jax.ShapeDtypeStruct((6, NPAD), _f32),
                   jax.ShapeDtypeStruct((NPAD, HID), _f32),
                   jax.ShapeDtypeStruct((NPAD, HID), _f32),
                   jax.ShapeDtypeStruct((NPAD, HID), _f32),
                   jax.ShapeDtypeStruct((NPAD,), _f32),
                   jax.ShapeDtypeStruct((NPAD,), _f32),
                   jax.ShapeDtypeStruct((NPAD,), _f32)),
    )(degp, xpad, w1)

    aggp = _sc_agg1(*es, y0, y1, y2, znk)

    z0, z1, z2 = pl.pallas_call(
        _tc2_body,
        out_shape=(jax.ShapeDtypeStruct((NPAD, 1), _f32),
                   jax.ShapeDtypeStruct((NPAD, 1), _f32),
                   jax.ShapeDtypeStruct((NPAD, 1), _f32)),
    )(aggp, dinv, b1, w2)

    outp = _sc_agg2(*es, z0.reshape(NPAD), z1.reshape(NPAD),
                    z2.reshape(NPAD), di0, di1, di2, zn)
    out = outp[0] + outp[1] + b2[0, 0]
    return out[:N]


# speedup vs baseline: 42.3343x; 1.0029x over previous
import functools

import jax
import jax.numpy as jnp
from jax import lax
from jax.experimental import pallas as pl
from jax.experimental.pallas import tpu as pltpu
from jax.experimental.pallas import tpu_sc as plsc

N = 10000
E = 320000
IN_F = 128
HID = 16
NPAD = 10240
NC = 2
NS = 16
NW = NC * NS
EW = E // NW
CH = 2000
NCH = EW // CH
SL = NPAD // NS

_mesh = plsc.VectorSubcoreMesh(
    core_axis_name="c", subcore_axis_name="s", num_cores=NC, num_subcores=NS)

_f32 = jnp.float32
_SC_PARAMS = pltpu.CompilerParams(use_tc_tiling_on_sc=False,
                                  needs_layout_passes=False)


@functools.partial(
    pl.kernel,
    out_type=jax.ShapeDtypeStruct((NC, 6, NPAD), _f32),
    mesh=_mesh,
    compiler_params=_SC_PARAMS,
    scratch_types=(
        [pltpu.VMEM((EW,), jnp.int32), pltpu.VMEM((EW,), _f32)]
        + [pltpu.VMEM_SHARED((NPAD,), _f32) for _ in range(6)]
    ),
)
def _sc_degrees(e0, e1, e2, zn, ones_hbm, out, idx_v, ones_v, *hists):
    cid = lax.axis_index("c")
    sid = lax.axis_index("s")
    wid = sid * NC + cid
    for h in hists:
        pltpu.sync_copy(zn.at[pl.ds(sid * SL, SL)], h.at[pl.ds(sid * SL, SL)])
    pltpu.sync_copy(ones_hbm, ones_v)
    plsc.subcore_barrier()
    for t, ei in enumerate((e0, e1, e2)):
        for r in range(2):
            pltpu.sync_copy(ei.at[pl.ds(r * E + wid * EW, EW)], idx_v)
            pltpu.sync_copy(ones_v, hists[2 * t + r].at[idx_v], add=True)
    plsc.subcore_barrier()
    for hi, h in enumerate(hists):
        pltpu.sync_copy(h.at[pl.ds(sid * SL, SL)],
                        out.at[cid, hi, pl.ds(sid * SL, SL)])


@functools.partial(
    pl.kernel,
    out_type=jax.ShapeDtypeStruct((NC, 3, NPAD, HID), _f32),
    mesh=_mesh,
    compiler_params=_SC_PARAMS,
    scratch_types=(
        [pltpu.VMEM((CH,), jnp.int32) for _ in range(4)]
        + [pltpu.VMEM((CH, HID), _f32) for _ in range(2)]
        + [pltpu.SemaphoreType.DMA for _ in range(4)]
        + [pltpu.VMEM_SHARED((NPAD, HID), _f32) for _ in range(3)]
    ),
)
def _sc_agg1(e0, e1, e2, y0, y1, y2, znk, out,
             sidx0, sidx1, didx0, didx1, rows0, rows1,
             gsem0, gsem1, ssem0, ssem1, *accs):
    cid = lax.axis_index("c")
    sid = lax.axis_index("s")
    wid = sid * NC + cid
    sidx = (sidx0, sidx1)
    didx = (didx0, didx1)
    rows = (rows0, rows1)
    gsem = (gsem0, gsem1)
    ssem = (ssem0, ssem1)
    for a in accs:
        pltpu.sync_copy(znk.at[pl.ds(sid * SL, SL)], a.at[pl.ds(sid * SL, SL)])
    plsc.subcore_barrier()
    ys = (y0, y1, y2)
    eis = (e0, e1, e2)
    chunks = [(t, k) for t in range(3) for k in range(NCH)]
    tot = len(chunks)

    def load_idx(cnt, b):
        t, k = chunks[cnt]
        base = wid * EW + k * CH
        pltpu.sync_copy(eis[t].at[pl.ds(base, CH)], sidx[b])
        pltpu.sync_copy(eis[t].at[pl.ds(E + base, CH)], didx[b])

    descs_g = [None] * tot
    load_idx(0, 0)
    descs_g[0] = pltpu.async_copy(ys[chunks[0][0]].at[sidx[0]], rows[0], gsem[0])
    for cnt in range(tot):
        b = cnt % 2
        nb = 1 - b
        if cnt + 1 < tot:
            load_idx(cnt + 1, nb)
            descs_g[cnt + 1] = pltpu.async_copy(
                ys[chunks[cnt + 1][0]].at[sidx[nb]], rows[nb], gsem[nb])
        descs_g[cnt].wait()
        pltpu.sync_copy(rows[b], accs[chunks[cnt][0]].at[didx[b]], add=True)
    plsc.subcore_barrier()
    for t, a in enumerate(accs):
        pltpu.sync_copy(a.at[pl.ds(sid * SL, SL)],
                        out.at[cid, t, pl.ds(sid * SL, SL)])


@functools.partial(
    pl.kernel,
    out_type=jax.ShapeDtypeStruct((NC, NPAD), _f32),
    mesh=_mesh,
    compiler_params=_SC_PARAMS,
    scratch_types=(
        [pltpu.VMEM((NPAD,), _f32) for _ in range(6)]
        + [pltpu.VMEM((CH,), jnp.int32) for _ in range(3)]
        + [pltpu.VMEM((CH,), _f32) for _ in range(2)]
        + [pltpu.SemaphoreType.DMA for _ in range(2)]
        + [pltpu.VMEM_SHARED((NPAD,), _f32)]
    ),
)
def _sc_agg2(e0, e1, e2, z0, z1, z2, di0, di1, di2, zn, out,
             zv0, zv1, zv2, dv0, dv1, dv2, sidx, didx0, didx1,
             vals0, vals1, ssem0, ssem1, acc):
    cid = lax.axis_index("c")
    sid = lax.axis_index("s")
    wid = sid * NC + cid
    didx = (didx0, didx1)
    vals = (vals0, vals1)
    ssem = (ssem0, ssem1)
    pltpu.sync_copy(zn.at[pl.ds(sid * SL, SL)], acc.at[pl.ds(sid * SL, SL)])
    for hbm, v in ((z0, zv0), (z1, zv1), (z2, zv2),
                   (di0, dv0), (di1, dv1), (di2, dv2)):
        pltpu.sync_copy(hbm, v)
    plsc.subcore_barrier()
    zvs = (zv0, zv1, zv2)
    dvs = (dv0, dv1, dv2)
    eis = (e0, e1, e2)
    chunks = [(t, k) for t in range(3) for k in range(NCH)]
    tot = len(chunks)
    descs_s = [None] * tot
    for cnt in range(tot):
        t, k = chunks[cnt]
        b = cnt % 2
        base = wid * EW + k * CH
        pltpu.sync_copy(eis[t].at[pl.ds(base, CH)], sidx)
        pltpu.sync_copy(eis[t].at[pl.ds(E + base, CH)], didx[b])
        zv = zvs[t]
        dv = dvs[t]
        vb = vals[b]
        db = didx[b]

        def body(i, _, zv=zv, dv=dv, vb=vb, db=db):
            idxs = sidx[pl.ds(i * 16, 16)]
            idxd = db[pl.ds(i * 16, 16)]
            v = plsc.load_gather(zv, [idxs]) * plsc.load_gather(dv, [idxd])
            vb[pl.ds(i * 16, 16)] = v
            return 0

        lax.fori_loop(0, CH // 16, body, 0, unroll=5)
        descs_s[cnt] = pltpu.async_copy(vb, acc.at[db], ssem[b], add=True)
        if cnt >= 1:
            descs_s[cnt - 1].wait()
    descs_s[tot - 1].wait()
    plsc.subcore_barrier()
    pltpu.sync_copy(acc.at[pl.ds(sid * SL, SL)],
                    out.at[cid, pl.ds(sid * SL, SL)])


def _tc1_body(degp_ref, x_ref, w_ref, dinv_ref, y0_ref, y1_ref, y2_ref,
              di0_ref, di1_ref, di2_ref):
    deg = jnp.maximum(degp_ref[0] + degp_ref[1], 1.0)
    dinv = lax.rsqrt(deg)
    dinv_ref[...] = dinv
    y = jnp.dot(x_ref[...], w_ref[...], preferred_element_type=_f32)
    for e, yr in enumerate((y0_ref, y1_ref, y2_ref)):
        yr[...] = y[:, 16 * e:16 * (e + 1)] * dinv[2 * e][:, None]
    for e, dr in enumerate((di0_ref, di1_ref, di2_ref)):
        dr[...] = dinv[2 * e + 1]


def _tc2_body(aggp_ref, dinv_ref, b1_ref, w2_ref, z0_ref, z1_ref, z2_ref):
    dinv = dinv_ref[...]
    h = jnp.zeros((NPAD, HID), _f32)
    for e in range(3):
        h = h + (aggp_ref[0, e] + aggp_ref[1, e]) * dinv[2 * e + 1][:, None]
    h = jnp.maximum(h + jnp.sum(b1_ref[...], axis=0)[None, :], 0.0)
    for e, zr in enumerate((z0_ref, z1_ref, z2_ref)):
        zr[...] = jnp.dot(h, w2_ref[:, e:e + 1],
                          preferred_element_type=_f32) * dinv[2 * e][:, None]


def kernel(x, edge_index_rsr, edge_index_rtr, edge_index_rur,
           W1_rsr, b1_rsr, W1_rtr, b1_rtr, W1_rur, b1_rur,
           W2_rsr, b2_rsr, W2_rtr, b2_rtr, W2_rur, b2_rur):
    xpad = jnp.zeros((NPAD, IN_F), _f32).at[:N].set(x)
    w1 = jnp.concatenate([W1_rsr, W1_rtr, W1_rur], axis=1)
    w2 = jnp.concatenate([W2_rsr, W2_rtr, W2_rur], axis=1)
    b1 = jnp.stack([b1_rsr, b1_rtr, b1_rur])
    b2 = (b2_rsr + b2_rtr + b2_rur).reshape(1, 1)
    zn = jnp.zeros((NPAD,), _f32)
    znk = jnp.zeros((NPAD, HID), _f32)
    ones = jnp.ones((EW,), _f32)
    es = (edge_index_rsr.reshape(2 * E), edge_index_rtr.reshape(2 * E),
          edge_index_rur.reshape(2 * E))

    degp = _sc_degrees(*es, zn, ones)

    dinv, y0, y1, y2, di0, di1, di2 = pl.pallas_call(
        _tc1_body,
        out_shape=(jax.ShapeDtypeStruct((6, NPAD), _f32),
                   jax.ShapeDtypeStruct((NPAD, HID), _f32),
                   jax.ShapeDtypeStruct((NPAD, HID), _f32),
                   jax.ShapeDtypeStruct((NPAD, HID), _f32),
                   jax.ShapeDtypeStruct((NPAD,), _f32),
                   jax.ShapeDtypeStruct((NPAD,), _f32),
                   jax.ShapeDtypeStruct((NPAD,), _f32)),
    )(degp, xpad, w1)

    aggp = _sc_agg1(*es, y0, y1, y2, znk)

    z0, z1, z2 = pl.pallas_call(
        _tc2_body,
        out_shape=(jax.ShapeDtypeStruct((NPAD, 1), _f32),
                   jax.ShapeDtypeStruct((NPAD, 1), _f32),
                   jax.ShapeDtypeStruct((NPAD, 1), _f32)),
    )(aggp, dinv, b1, w2)

    outp = _sc_agg2(*es, z0.reshape(NPAD), z1.reshape(NPAD),
                    z2.reshape(NPAD), di0, di1, di2, zn)
    out = outp[0] + outp[1] + b2[0, 0]
    return out[:N]


# speedup vs baseline: 43.1866x; 1.0201x over previous
import functools

import jax
import jax.numpy as jnp
from jax import lax
from jax.experimental import pallas as pl
from jax.experimental.pallas import tpu as pltpu
from jax.experimental.pallas import tpu_sc as plsc

N = 10000
E = 320000
IN_F = 128
HID = 16
NPAD = 10240
NP8 = NPAD // 8
NC = 2
NS = 16
NW = NC * NS
GW = 78
EWM = GW * 128
CW = 1664
NCH = EWM // CW
TB = 2496 * 128
SL = NPAD // NS

_mesh = plsc.VectorSubcoreMesh(
    core_axis_name="c", subcore_axis_name="s", num_cores=NC, num_subcores=NS)

_f32 = jnp.float32
_SC_PARAMS = pltpu.CompilerParams(use_tc_tiling_on_sc=False,
                                  needs_layout_passes=False)


@functools.partial(
    pl.kernel,
    out_type=jax.ShapeDtypeStruct((NC, 6, NPAD), _f32),
    mesh=_mesh,
    compiler_params=_SC_PARAMS,
    scratch_types=(
        [pltpu.VMEM((EWM,), jnp.int32), pltpu.VMEM((128,), jnp.int32),
         pltpu.VMEM((EWM,), _f32)]
        + [pltpu.VMEM_SHARED((NPAD,), _f32) for _ in range(6)]
    ),
)
def _sc_degrees(e0, e1, e2, zn, ones_hbm, out, idx_v, idx_t, ones_v, *hists):
    cid = lax.axis_index("c")
    sid = lax.axis_index("s")
    wid = sid * NC + cid
    for h in hists:
        pltpu.sync_copy(zn.at[pl.ds(sid * SL, SL)], h.at[pl.ds(sid * SL, SL)])
    pltpu.sync_copy(ones_hbm, ones_v)
    plsc.subcore_barrier()
    for t, ei in enumerate((e0, e1, e2)):
        for r in range(2):
            pltpu.sync_copy(ei.at[r, pl.ds(wid * EWM, EWM)], idx_v)
            pltpu.sync_copy(ones_v, hists[2 * t + r].at[idx_v], add=True)

            @pl.when(wid < 4)
            def _(ei=ei, r=r, h=hists[2 * t + r]):
                pltpu.sync_copy(ei.at[r, pl.ds(TB + wid * 128, 128)], idx_t)
                pltpu.sync_copy(ones_v.at[pl.ds(0, 128)], h.at[idx_t], add=True)

    plsc.subcore_barrier()
    for hi, h in enumerate(hists):
        pltpu.sync_copy(h.at[pl.ds(sid * SL, SL)],
                        out.at[cid, hi, pl.ds(sid * SL, SL)])


@functools.partial(
    pl.kernel,
    out_type=jax.ShapeDtypeStruct((NC, 3, NP8, 128), _f32),
    mesh=_mesh,
    compiler_params=_SC_PARAMS,
    scratch_types=(
        [pltpu.VMEM((CW,), jnp.int32) for _ in range(4)]
        + [pltpu.VMEM((CW, HID), _f32) for _ in range(2)]
        + [pltpu.VMEM((128,), jnp.int32) for _ in range(2)]
        + [pltpu.VMEM((128, HID), _f32)]
        + [pltpu.VMEM((SL, HID), _f32), pltpu.VMEM((SL // 8, 128), _f32)]
        + [pltpu.SemaphoreType.DMA for _ in range(3)]
        + [pltpu.VMEM_SHARED((NPAD, HID), _f32) for _ in range(3)]
    ),
)
def _sc_agg1(e0, e1, e2, y0, y1, y2, znk, out,
             sidx0, sidx1, didx0, didx1, rows0, rows1,
             sidx_t, didx_t, rows_t, bufa, bufb,
             gsem0, gsem1, tsem, *accs):
    cid = lax.axis_index("c")
    sid = lax.axis_index("s")
    wid = sid * NC + cid
    sidx = (sidx0, sidx1)
    didx = (didx0, didx1)
    rows = (rows0, rows1)
    gsem = (gsem0, gsem1)
    for a in accs:
        pltpu.sync_copy(znk.at[pl.ds(sid * SL, SL)], a.at[pl.ds(sid * SL, SL)])
    plsc.subcore_barrier()
    ys = (y0, y1, y2)
    eis = (e0, e1, e2)
    chunks = [(t, k) for t in range(3) for k in range(NCH)]
    tot = len(chunks)

    def load_idx(cnt, b):
        t, k = chunks[cnt]
        base = wid * EWM + k * CW
        pltpu.sync_copy(eis[t].at[0, pl.ds(base, CW)], sidx[b])
        pltpu.sync_copy(eis[t].at[1, pl.ds(base, CW)], didx[b])

    descs_g = [None] * tot
    load_idx(0, 0)
    descs_g[0] = pltpu.async_copy(ys[chunks[0][0]].at[sidx[0]], rows[0], gsem[0])
    for cnt in range(tot):
        b = cnt % 2
        nb = 1 - b
        if cnt + 1 < tot:
            load_idx(cnt + 1, nb)
            descs_g[cnt + 1] = pltpu.async_copy(
                ys[chunks[cnt + 1][0]].at[sidx[nb]], rows[nb], gsem[nb])
        descs_g[cnt].wait()
        pltpu.sync_copy(rows[b], accs[chunks[cnt][0]].at[didx[b]], add=True)

    @pl.when(wid < 4)
    def _():
        for t in range(3):
            pltpu.sync_copy(eis[t].at[0, pl.ds(TB + wid * 128, 128)], sidx_t)
            pltpu.sync_copy(eis[t].at[1, pl.ds(TB + wid * 128, 128)], didx_t)
            pltpu.async_copy(ys[t].at[sidx_t], rows_t, tsem).wait()
            pltpu.sync_copy(rows_t, accs[t].at[didx_t], add=True)

    plsc.subcore_barrier()
    for t, a in enumerate(accs):
        pltpu.sync_copy(a.at[pl.ds(sid * SL, SL)], bufa)

        def repack(r, _):
            bufb[r // 8, pl.ds((r % 8) * HID, HID)] = bufa[r, :]
            return 0

        lax.fori_loop(0, SL, repack, 0, unroll=8)
        pltpu.sync_copy(bufb, out.at[cid, t, pl.ds(sid * (SL // 8), SL // 8)])


@functools.partial(
    pl.kernel,
    out_type=jax.ShapeDtypeStruct((NC, NPAD), _f32),
    mesh=_mesh,
    compiler_params=_SC_PARAMS,
    scratch_types=(
        [pltpu.VMEM((NPAD,), _f32) for _ in range(9)]
        + [pltpu.VMEM((CW,), jnp.int32) for _ in range(3)]
        + [pltpu.VMEM((CW,), _f32) for _ in range(2)]
        + [pltpu.VMEM((128,), jnp.int32) for _ in range(2)]
        + [pltpu.VMEM((128,), _f32)]
        + [pltpu.SemaphoreType.DMA for _ in range(2)]
        + [pltpu.VMEM_SHARED((NPAD,), _f32)]
    ),
)
def _sc_agg2(e0, e1, e2, z0, z1, z2, di0, di1, di2, do0, do1, do2, zn, out,
             zv0, zv1, zv2, dv0, dv1, dv2, ov0, ov1, ov2, sidx, didx0, didx1,
             vals0, vals1, sidx_t, didx_t, vals_t, ssem0, ssem1, acc):
    cid = lax.axis_index("c")
    sid = lax.axis_index("s")
    wid = sid * NC + cid
    didx = (didx0, didx1)
    vals = (vals0, vals1)
    ssem = (ssem0, ssem1)
    pltpu.sync_copy(zn.at[pl.ds(sid * SL, SL)], acc.at[pl.ds(sid * SL, SL)])
    for hbm, v in ((z0, zv0), (z1, zv1), (z2, zv2),
                   (di0, dv0), (di1, dv1), (di2, dv2),
                   (do0, ov0), (do1, ov1), (do2, ov2)):
        pltpu.sync_copy(hbm, v)
    for zv, ov in ((zv0, ov0), (zv1, ov1), (zv2, ov2)):

        def scale(i, _, zv=zv, ov=ov):
            s = pl.ds(i * 16, 16)
            zv[s] = zv[s] * ov[s]
            return 0

        lax.fori_loop(0, NPAD // 16, scale, 0, unroll=8)
    plsc.subcore_barrier()
    zvs = (zv0, zv1, zv2)
    dvs = (dv0, dv1, dv2)
    eis = (e0, e1, e2)
    chunks = [(t, k) for t in range(3) for k in range(NCH)]
    tot = len(chunks)
    descs_s = [None] * tot
    for cnt in range(tot):
        t, k = chunks[cnt]
        b = cnt % 2
        base = wid * EWM + k * CW
        pltpu.sync_copy(eis[t].at[0, pl.ds(base, CW)], sidx)
        pltpu.sync_copy(eis[t].at[1, pl.ds(base, CW)], didx[b])
        zv = zvs[t]
        dv = dvs[t]
        vb = vals[b]
        db = didx[b]

        def body(i, _, zv=zv, dv=dv, vb=vb, db=db):
            idxs = sidx[pl.ds(i * 16, 16)]
            idxd = db[pl.ds(i * 16, 16)]
            v = plsc.load_gather(zv, [idxs]) * plsc.load_gather(dv, [idxd])
            vb[pl.ds(i * 16, 16)] = v
            return 0

        lax.fori_loop(0, CW // 16, body, 0, unroll=8)
        descs_s[cnt] = pltpu.async_copy(vb, acc.at[db], ssem[b], add=True)
        if cnt >= 1:
            descs_s[cnt - 1].wait()
    descs_s[tot - 1].wait()

    @pl.when(wid < 4)
    def _():
        for t in range(3):
            pltpu.sync_copy(eis[t].at[0, pl.ds(TB + wid * 128, 128)], sidx_t)
            pltpu.sync_copy(eis[t].at[1, pl.ds(TB + wid * 128, 128)], didx_t)

            def body_t(i, _, t=t):
                idxs = sidx_t[pl.ds(i * 16, 16)]
                idxd = didx_t[pl.ds(i * 16, 16)]
                v = plsc.load_gather(zvs[t], [idxs]) * plsc.load_gather(dvs[t], [idxd])
                vals_t[pl.ds(i * 16, 16)] = v
                return 0

            lax.fori_loop(0, 8, body_t, 0, unroll=8)
            pltpu.sync_copy(vals_t, acc.at[didx_t], add=True)

    plsc.subcore_barrier()
    pltpu.sync_copy(acc.at[pl.ds(sid * SL, SL)],
                    out.at[cid, pl.ds(sid * SL, SL)])


def _tc1_body(degp_ref, x_ref, w_ref, dinv_ref, y0_ref, y1_ref, y2_ref,
              di0_ref, di1_ref, di2_ref, do0_ref, do1_ref, do2_ref):
    deg = jnp.maximum(degp_ref[0] + degp_ref[1], 1.0)
    dinv = lax.rsqrt(deg)
    dinv_ref[...] = dinv
    y = jnp.dot(x_ref[...], w_ref[...], preferred_element_type=_f32)
    for e, yr in enumerate((y0_ref, y1_ref, y2_ref)):
        yr[...] = y[:, 16 * e:16 * (e + 1)] * dinv[2 * e][:, None]
    for e, dr in enumerate((di0_ref, di1_ref, di2_ref)):
        dr[...] = dinv[2 * e + 1]
    for e, dr in enumerate((do0_ref, do1_ref, do2_ref)):
        dr[...] = dinv[2 * e]


def _tc2_body(aggp_ref, dr0_ref, dr1_ref, dr2_ref, b1t_ref, w2blk_ref,
              z0_ref, z1_ref, z2_ref):
    hp = jnp.zeros((NP8, 128), _f32)
    for e, dr in enumerate((dr0_ref, dr1_ref, dr2_ref)):
        hp = hp + (aggp_ref[0, e] + aggp_ref[1, e]) * dr[...]
    hp = jnp.maximum(hp + b1t_ref[0][None, :], 0.0)
    zp = jnp.dot(hp, w2blk_ref[...], preferred_element_type=_f32)
    for e, zr in enumerate((z0_ref, z1_ref, z2_ref)):
        zr[...] = zp[:, 8 * e:8 * (e + 1)]


def kernel(x, edge_index_rsr, edge_index_rtr, edge_index_rur,
           W1_rsr, b1_rsr, W1_rtr, b1_rtr, W1_rur, b1_rur,
           W2_rsr, b2_rsr, W2_rtr, b2_rtr, W2_rur, b2_rur):
    xpad = jnp.zeros((NPAD, IN_F), _f32).at[:N].set(x)
    w1 = jnp.concatenate([W1_rsr, W1_rtr, W1_rur], axis=1)
    w2 = jnp.concatenate([W2_rsr, W2_rtr, W2_rur], axis=1)
    b1t = jnp.tile(b1_rsr + b1_rtr + b1_rur, 8).reshape(1, 128)
    b2s = b2_rsr + b2_rtr + b2_rur
    w2blk = jnp.zeros((128, 24), _f32)
    for j in range(8):
        for e in range(3):
            w2blk = w2blk.at[16 * j:16 * (j + 1), 8 * e + j].set(w2[:, e])
    zn = jnp.zeros((NPAD,), _f32)
    znk = jnp.zeros((NPAD, HID), _f32)
    ones = jnp.ones((EWM,), _f32)
    es = (edge_index_rsr, edge_index_rtr, edge_index_rur)

    degp = _sc_degrees(*es, zn, ones)

    (dinv, y0, y1, y2, di0, di1, di2, do0, do1, do2) = pl.pallas_call(
        _tc1_body,
        out_shape=(jax.ShapeDtypeStruct((6, NPAD), _f32),)
        + (jax.ShapeDtypeStruct((NPAD, HID), _f32),) * 3
        + (jax.ShapeDtypeStruct((NPAD,), _f32),) * 6,
    )(degp, xpad, w1)

    aggp = _sc_agg1(*es, y0, y1, y2, znk)

    drep = [jnp.repeat(d, 16).reshape(NP8, 128) for d in (di0, di1, di2)]
    zp0, zp1, zp2 = pl.pallas_call(
        _tc2_body,
        out_shape=(jax.ShapeDtypeStruct((NP8, 8), _f32),) * 3,
    )(aggp, *drep, b1t, w2blk)

    outp = _sc_agg2(*es, zp0.reshape(NPAD), zp1.reshape(NPAD),
                    zp2.reshape(NPAD), di0, di1, di2,
                    do0, do1, do2, zn)
    out = outp[0] + outp[1] + b2s[0]
    return out[:N]


# speedup vs baseline: 47.4180x; 1.0980x over previous
import functools

import jax
import jax.numpy as jnp
from jax import lax
from jax.experimental import pallas as pl
from jax.experimental.pallas import tpu as pltpu
from jax.experimental.pallas import tpu_sc as plsc

N = 10000
E = 320000
IN_F = 128
HID = 16
NPAD = 10240
NP8 = NPAD // 8
NC = 2
NS = 16
NW = NC * NS
GW = 78
EWM = GW * 128
CW1 = 1664
NCH1 = EWM // CW1
CW = 2496
NCH = EWM // CW
TB = 2496 * 128
SL = NPAD // NS

_mesh = plsc.VectorSubcoreMesh(
    core_axis_name="c", subcore_axis_name="s", num_cores=NC, num_subcores=NS)

_f32 = jnp.float32
_SC_PARAMS = pltpu.CompilerParams(use_tc_tiling_on_sc=False,
                                  needs_layout_passes=False)


@functools.partial(
    pl.kernel,
    out_type=jax.ShapeDtypeStruct((NC, 6, NPAD), _f32),
    mesh=_mesh,
    compiler_params=_SC_PARAMS,
    scratch_types=(
        [pltpu.VMEM((EWM,), jnp.int32), pltpu.VMEM((EWM,), jnp.int32),
         pltpu.VMEM((128,), jnp.int32), pltpu.VMEM((EWM,), _f32)]
        + [pltpu.SemaphoreType.DMA for _ in range(2)]
        + [pltpu.VMEM_SHARED((NPAD,), _f32) for _ in range(6)]
    ),
)
def _sc_degrees(e0, e1, e2, zn, ones_hbm, out, idx0, idx1, idx_t, ones_v,
                hsem0, hsem1, *hists):
    cid = lax.axis_index("c")
    sid = lax.axis_index("s")
    wid = sid * NC + cid
    idxs = (idx0, idx1)
    hsem = (hsem0, hsem1)
    for h in hists:
        pltpu.sync_copy(zn.at[pl.ds(sid * SL, SL)], h.at[pl.ds(sid * SL, SL)])
    pltpu.sync_copy(ones_hbm, ones_v)
    plsc.subcore_barrier()
    units = [(t, r) for t in range(3) for r in range(2)]
    descs = [None] * 6
    eis = (e0, e1, e2)
    pltpu.sync_copy(eis[0].at[0, pl.ds(wid * EWM, EWM)], idx0)
    for cnt, (t, r) in enumerate(units):
        b = cnt % 2
        descs[cnt] = pltpu.async_copy(
            ones_v, hists[2 * t + r].at[idxs[b]], hsem[b], add=True)
        if cnt + 1 < 6:
            t2, r2 = units[cnt + 1]
            if cnt >= 1:
                descs[cnt - 1].wait()
            pltpu.sync_copy(eis[t2].at[r2, pl.ds(wid * EWM, EWM)], idxs[1 - b])
    descs[4].wait()
    descs[5].wait()

    @pl.when(wid < 4)
    def _():
        for t in range(3):
            for r in range(2):
                pltpu.sync_copy(eis[t].at[r, pl.ds(TB + wid * 128, 128)], idx_t)
                pltpu.sync_copy(ones_v.at[pl.ds(0, 128)],
                                hists[2 * t + r].at[idx_t], add=True)

    plsc.subcore_barrier()
    for hi, h in enumerate(hists):
        pltpu.sync_copy(h.at[pl.ds(sid * SL, SL)],
                        out.at[cid, hi, pl.ds(sid * SL, SL)])


@functools.partial(
    pl.kernel,
    out_type=jax.ShapeDtypeStruct((NC, 3, NP8, 128), _f32),
    mesh=_mesh,
    compiler_params=_SC_PARAMS,
    scratch_types=(
        [pltpu.VMEM((CW1,), jnp.int32) for _ in range(4)]
        + [pltpu.VMEM((CW1, HID), _f32) for _ in range(2)]
        + [pltpu.VMEM((128,), jnp.int32) for _ in range(2)]
        + [pltpu.VMEM((128, HID), _f32)]
        + [pltpu.VMEM((SL, HID), _f32), pltpu.VMEM((SL // 8, 128), _f32)]
        + [pltpu.SemaphoreType.DMA for _ in range(3)]
        + [pltpu.VMEM_SHARED((NPAD, HID), _f32) for _ in range(3)]
    ),
)
def _sc_agg1(e0, e1, e2, y0, y1, y2, znk, out,
             sidx0, sidx1, didx0, didx1, rows0, rows1,
             sidx_t, didx_t, rows_t, bufa, bufb,
             gsem0, gsem1, tsem, *accs):
    cid = lax.axis_index("c")
    sid = lax.axis_index("s")
    wid = sid * NC + cid
    sidx = (sidx0, sidx1)
    didx = (didx0, didx1)
    rows = (rows0, rows1)
    gsem = (gsem0, gsem1)
    for a in accs:
        pltpu.sync_copy(znk.at[pl.ds(sid * SL, SL)], a.at[pl.ds(sid * SL, SL)])
    plsc.subcore_barrier()
    ys = (y0, y1, y2)
    eis = (e0, e1, e2)
    chunks = [(t, k) for t in range(3) for k in range(NCH1)]
    tot = len(chunks)

    def load_idx(cnt, b):
        t, k = chunks[cnt]
        base = wid * EWM + k * CW1
        pltpu.sync_copy(eis[t].at[0, pl.ds(base, CW1)], sidx[b])
        pltpu.sync_copy(eis[t].at[1, pl.ds(base, CW1)], didx[b])

    descs_g = [None] * tot
    load_idx(0, 0)
    descs_g[0] = pltpu.async_copy(ys[chunks[0][0]].at[sidx[0]], rows[0], gsem[0])
    for cnt in range(tot):
        b = cnt % 2
        nb = 1 - b
        if cnt + 1 < tot:
            load_idx(cnt + 1, nb)
            descs_g[cnt + 1] = pltpu.async_copy(
                ys[chunks[cnt + 1][0]].at[sidx[nb]], rows[nb], gsem[nb])
        descs_g[cnt].wait()
        pltpu.sync_copy(rows[b], accs[chunks[cnt][0]].at[didx[b]], add=True)

    @pl.when(wid < 4)
    def _():
        for t in range(3):
            pltpu.sync_copy(eis[t].at[0, pl.ds(TB + wid * 128, 128)], sidx_t)
            pltpu.sync_copy(eis[t].at[1, pl.ds(TB + wid * 128, 128)], didx_t)
            pltpu.async_copy(ys[t].at[sidx_t], rows_t, tsem).wait()
            pltpu.sync_copy(rows_t, accs[t].at[didx_t], add=True)

    plsc.subcore_barrier()
    for t, a in enumerate(accs):
        pltpu.sync_copy(a.at[pl.ds(sid * SL, SL)], bufa)

        def repack(r, _):
            bufb[r // 8, pl.ds((r % 8) * HID, HID)] = bufa[r, :]
            return 0

        lax.fori_loop(0, SL, repack, 0, unroll=8)
        pltpu.sync_copy(bufb, out.at[cid, t, pl.ds(sid * (SL // 8), SL // 8)])


@functools.partial(
    pl.kernel,
    out_type=jax.ShapeDtypeStruct((NC, NPAD), _f32),
    mesh=_mesh,
    compiler_params=_SC_PARAMS,
    scratch_types=(
        [pltpu.VMEM((NPAD,), _f32) for _ in range(6)]
        + [pltpu.VMEM((CW,), jnp.int32) for _ in range(3)]
        + [pltpu.VMEM((CW,), _f32) for _ in range(2)]
        + [pltpu.VMEM((128,), jnp.int32) for _ in range(2)]
        + [pltpu.VMEM((128,), _f32)]
        + [pltpu.SemaphoreType.DMA for _ in range(2)]
        + [pltpu.VMEM_SHARED((NPAD,), _f32)]
    ),
)
def _sc_agg2(e0, e1, e2, z0, z1, z2, di0, di1, di2, zn, out,
             zv0, zv1, zv2, dv0, dv1, dv2, sidx, didx0, didx1,
             vals0, vals1, sidx_t, didx_t, vals_t, ssem0, ssem1, acc):
    cid = lax.axis_index("c")
    sid = lax.axis_index("s")
    wid = sid * NC + cid
    didx = (didx0, didx1)
    vals = (vals0, vals1)
    ssem = (ssem0, ssem1)
    pltpu.sync_copy(zn.at[pl.ds(sid * SL, SL)], acc.at[pl.ds(sid * SL, SL)])
    for hbm, v in ((z0, zv0), (z1, zv1), (z2, zv2),
                   (di0, dv0), (di1, dv1), (di2, dv2)):
        pltpu.sync_copy(hbm, v)
    plsc.subcore_barrier()
    zvs = (zv0, zv1, zv2)
    dvs = (dv0, dv1, dv2)
    eis = (e0, e1, e2)
    chunks = [(t, k) for t in range(3) for k in range(NCH)]
    tot = len(chunks)
    descs_s = [None] * tot
    for cnt in range(tot):
        t, k = chunks[cnt]
        b = cnt % 2
        base = wid * EWM + k * CW
        pltpu.sync_copy(eis[t].at[0, pl.ds(base, CW)], sidx)
        pltpu.sync_copy(eis[t].at[1, pl.ds(base, CW)], didx[b])
        zv = zvs[t]
        dv = dvs[t]
        vb = vals[b]
        db = didx[b]

        def body(i, _, zv=zv, dv=dv, vb=vb, db=db):
            idxs = sidx[pl.ds(i * 16, 16)]
            idxd = db[pl.ds(i * 16, 16)]
            v = plsc.load_gather(zv, [idxs]) * plsc.load_gather(dv, [idxd])
            vb[pl.ds(i * 16, 16)] = v
            return 0

        lax.fori_loop(0, CW // 16, body, 0, unroll=8)
        descs_s[cnt] = pltpu.async_copy(vb, acc.at[db], ssem[b], add=True)
        if cnt >= 1:
            descs_s[cnt - 1].wait()
    descs_s[tot - 1].wait()

    @pl.when(wid < 4)
    def _():
        for t in range(3):
            pltpu.sync_copy(eis[t].at[0, pl.ds(TB + wid * 128, 128)], sidx_t)
            pltpu.sync_copy(eis[t].at[1, pl.ds(TB + wid * 128, 128)], didx_t)

            def body_t(i, _, t=t):
                idxs = sidx_t[pl.ds(i * 16, 16)]
                idxd = didx_t[pl.ds(i * 16, 16)]
                v = plsc.load_gather(zvs[t], [idxs]) * plsc.load_gather(dvs[t], [idxd])
                vals_t[pl.ds(i * 16, 16)] = v
                return 0

            lax.fori_loop(0, 8, body_t, 0, unroll=8)
            pltpu.sync_copy(vals_t, acc.at[didx_t], add=True)

    plsc.subcore_barrier()
    pltpu.sync_copy(acc.at[pl.ds(sid * SL, SL)],
                    out.at[cid, pl.ds(sid * SL, SL)])


def _tc1_body(degp_ref, x_ref, w_ref, dinv_ref, y0_ref, y1_ref, y2_ref,
              di0_ref, di1_ref, di2_ref, do0_ref, do1_ref, do2_ref):
    deg = jnp.maximum(degp_ref[0] + degp_ref[1], 1.0)
    dinv = lax.rsqrt(deg)
    dinv_ref[...] = dinv
    y = jnp.dot(x_ref[...], w_ref[...], preferred_element_type=_f32)
    for e, yr in enumerate((y0_ref, y1_ref, y2_ref)):
        yr[...] = y[:, 16 * e:16 * (e + 1)] * dinv[2 * e][:, None]
    for e, dr in enumerate((di0_ref, di1_ref, di2_ref)):
        dr[...] = dinv[2 * e + 1]
    for e, dr in enumerate((do0_ref, do1_ref, do2_ref)):
        dr[...] = dinv[2 * e]


def _tc2_body(aggp_ref, dr0_ref, dr1_ref, dr2_ref, b1t_ref, w2blk_ref,
              z0_ref, z1_ref, z2_ref):
    hp = jnp.zeros((NP8, 128), _f32)
    for e, dr in enumerate((dr0_ref, dr1_ref, dr2_ref)):
        hp = hp + (aggp_ref[0, e] + aggp_ref[1, e]) * dr[...]
    hp = jnp.maximum(hp + b1t_ref[0][None, :], 0.0)
    zp = jnp.dot(hp, w2blk_ref[...], preferred_element_type=_f32)
    for e, zr in enumerate((z0_ref, z1_ref, z2_ref)):
        zr[...] = zp[:, 8 * e:8 * (e + 1)]


def kernel(x, edge_index_rsr, edge_index_rtr, edge_index_rur,
           W1_rsr, b1_rsr, W1_rtr, b1_rtr, W1_rur, b1_rur,
           W2_rsr, b2_rsr, W2_rtr, b2_rtr, W2_rur, b2_rur):
    xpad = jnp.zeros((NPAD, IN_F), _f32).at[:N].set(x)
    w1 = jnp.concatenate([W1_rsr, W1_rtr, W1_rur], axis=1)
    w2 = jnp.concatenate([W2_rsr, W2_rtr, W2_rur], axis=1)
    b1t = jnp.tile(b1_rsr + b1_rtr + b1_rur, 8).reshape(1, 128)
    b2s = b2_rsr + b2_rtr + b2_rur
    eye8 = jnp.eye(8, dtype=_f32)
    w2blk = jnp.concatenate(
        [jnp.kron(eye8, w2[:, e:e + 1]) for e in range(3)], axis=1)
    zn = jnp.zeros((NPAD,), _f32)
    znk = jnp.zeros((NPAD, HID), _f32)
    ones = jnp.ones((EWM,), _f32)
    es = (edge_index_rsr, edge_index_rtr, edge_index_rur)

    degp = _sc_degrees(*es, zn, ones)

    (dinv, y0, y1, y2, di0, di1, di2, do0, do1, do2) = pl.pallas_call(
        _tc1_body,
        out_shape=(jax.ShapeDtypeStruct((6, NPAD), _f32),)
        + (jax.ShapeDtypeStruct((NPAD, HID), _f32),) * 3
        + (jax.ShapeDtypeStruct((NPAD,), _f32),) * 6,
    )(degp, xpad, w1)

    aggp = _sc_agg1(*es, y0, y1, y2, znk)

    drep = [jnp.repeat(d, 16).reshape(NP8, 128) for d in (di0, di1, di2)]
    zp0, zp1, zp2 = pl.pallas_call(
        _tc2_body,
        out_shape=(jax.ShapeDtypeStruct((NP8, 8), _f32),) * 3,
    )(aggp, *drep, b1t, w2blk)

    outp = _sc_agg2(*es, zp0.reshape(NPAD) * do0, zp1.reshape(NPAD) * do1,
                    zp2.reshape(NPAD) * do2, di0, di1, di2, zn)
    out = outp[0] + outp[1] + b2s[0]
    return out[:N]


# speedup vs baseline: 49.8216x; 1.0507x over previous
import functools

import jax
import jax.numpy as jnp
from jax import lax
from jax.experimental import pallas as pl
from jax.experimental.pallas import tpu as pltpu
from jax.experimental.pallas import tpu_sc as plsc

N = 10000
E = 320000
IN_F = 128
HID = 16
NPAD = 10240
NP8 = NPAD // 8
NC = 2
NS = 16
NW = NC * NS
EWM = E // NW
CW1 = 2000
NCH1 = EWM // CW1
CW = 2000
NCH = EWM // CW
SL = NPAD // NS

_mesh = plsc.VectorSubcoreMesh(
    core_axis_name="c", subcore_axis_name="s", num_cores=NC, num_subcores=NS)

_f32 = jnp.float32
_SC_PARAMS = pltpu.CompilerParams(use_tc_tiling_on_sc=False,
                                  needs_layout_passes=False)


@functools.partial(
    pl.kernel,
    out_type=jax.ShapeDtypeStruct((NC, 6, NPAD), _f32),
    mesh=_mesh,
    compiler_params=_SC_PARAMS,
    scratch_types=(
        [pltpu.VMEM((EWM,), jnp.int32), pltpu.VMEM((EWM,), jnp.int32),
         pltpu.VMEM((EWM,), _f32)]
        + [pltpu.SemaphoreType.DMA for _ in range(2)]
        + [pltpu.VMEM_SHARED((NPAD,), _f32) for _ in range(6)]
    ),
)
def _sc_degrees(e0, e1, e2, zn, ones_hbm, out, idx0, idx1, ones_v,
                hsem0, hsem1, *hists):
    cid = lax.axis_index("c")
    sid = lax.axis_index("s")
    wid = sid * NC + cid
    idxs = (idx0, idx1)
    hsem = (hsem0, hsem1)
    for h in hists:
        pltpu.sync_copy(zn.at[pl.ds(sid * SL, SL)], h.at[pl.ds(sid * SL, SL)])
    pltpu.sync_copy(ones_hbm, ones_v)
    plsc.subcore_barrier()
    units = [(t, r) for t in range(3) for r in range(2)]
    descs = [None] * 6
    eis = (e0, e1, e2)
    pltpu.sync_copy(eis[0].at[0, pl.ds(wid * EWM, EWM)], idx0)
    for cnt, (t, r) in enumerate(units):
        b = cnt % 2
        descs[cnt] = pltpu.async_copy(
            ones_v, hists[2 * t + r].at[idxs[b]], hsem[b], add=True)
        if cnt + 1 < 6:
            t2, r2 = units[cnt + 1]
            if cnt >= 1:
                descs[cnt - 1].wait()
            pltpu.sync_copy(eis[t2].at[r2, pl.ds(wid * EWM, EWM)], idxs[1 - b])
    descs[4].wait()
    descs[5].wait()
    plsc.subcore_barrier()
    for hi, h in enumerate(hists):
        pltpu.sync_copy(h.at[pl.ds(sid * SL, SL)],
                        out.at[cid, hi, pl.ds(sid * SL, SL)])


@functools.partial(
    pl.kernel,
    out_type=jax.ShapeDtypeStruct((NC, 3, NP8, 128), _f32),
    mesh=_mesh,
    compiler_params=_SC_PARAMS,
    scratch_types=(
        [pltpu.VMEM((CW1,), jnp.int32) for _ in range(4)]
        + [pltpu.VMEM((CW1, HID), _f32) for _ in range(2)]
        + [pltpu.VMEM((SL, HID), _f32), pltpu.VMEM((SL // 8, 128), _f32)]
        + [pltpu.SemaphoreType.DMA for _ in range(2)]
        + [pltpu.VMEM_SHARED((NPAD, HID), _f32) for _ in range(3)]
    ),
)
def _sc_agg1(e0, e1, e2, y0, y1, y2, znk, out,
             sidx0, sidx1, didx0, didx1, rows0, rows1, bufa, bufb,
             gsem0, gsem1, *accs):
    cid = lax.axis_index("c")
    sid = lax.axis_index("s")
    wid = sid * NC + cid
    sidx = (sidx0, sidx1)
    didx = (didx0, didx1)
    rows = (rows0, rows1)
    gsem = (gsem0, gsem1)
    for a in accs:
        pltpu.sync_copy(znk.at[pl.ds(sid * SL, SL)], a.at[pl.ds(sid * SL, SL)])
    plsc.subcore_barrier()
    ys = (y0, y1, y2)
    eis = (e0, e1, e2)
    chunks = [(t, k) for t in range(3) for k in range(NCH1)]
    tot = len(chunks)

    def load_idx(cnt, b):
        t, k = chunks[cnt]
        base = wid * EWM + k * CW1
        pltpu.sync_copy(eis[t].at[0, pl.ds(base, CW1)], sidx[b])
        pltpu.sync_copy(eis[t].at[1, pl.ds(base, CW1)], didx[b])

    descs_g = [None] * tot
    load_idx(0, 0)
    descs_g[0] = pltpu.async_copy(ys[chunks[0][0]].at[sidx[0]], rows[0], gsem[0])
    for cnt in range(tot):
        b = cnt % 2
        nb = 1 - b
        if cnt + 1 < tot:
            load_idx(cnt + 1, nb)
            descs_g[cnt + 1] = pltpu.async_copy(
                ys[chunks[cnt + 1][0]].at[sidx[nb]], rows[nb], gsem[nb])
        descs_g[cnt].wait()
        pltpu.sync_copy(rows[b], accs[chunks[cnt][0]].at[didx[b]], add=True)

    plsc.subcore_barrier()
    for t, a in enumerate(accs):
        pltpu.sync_copy(a.at[pl.ds(sid * SL, SL)], bufa)

        def repack(r, _):
            bufb[r // 8, pl.ds((r % 8) * HID, HID)] = bufa[r, :]
            return 0

        lax.fori_loop(0, SL, repack, 0, unroll=8)
        pltpu.sync_copy(bufb, out.at[cid, t, pl.ds(sid * (SL // 8), SL // 8)])


@functools.partial(
    pl.kernel,
    out_type=jax.ShapeDtypeStruct((NC, NPAD), _f32),
    mesh=_mesh,
    compiler_params=_SC_PARAMS,
    scratch_types=(
        [pltpu.VMEM((NPAD,), _f32) for _ in range(6)]
        + [pltpu.VMEM((CW,), jnp.int32) for _ in range(3)]
        + [pltpu.VMEM((CW,), _f32) for _ in range(2)]
        + [pltpu.SemaphoreType.DMA for _ in range(2)]
        + [pltpu.VMEM_SHARED((NPAD,), _f32)]
    ),
)
def _sc_agg2(e0, e1, e2, z0, z1, z2, di0, di1, di2, zn, out,
             zv0, zv1, zv2, dv0, dv1, dv2, sidx, didx0, didx1,
             vals0, vals1, ssem0, ssem1, acc):
    cid = lax.axis_index("c")
    sid = lax.axis_index("s")
    wid = sid * NC + cid
    didx = (didx0, didx1)
    vals = (vals0, vals1)
    ssem = (ssem0, ssem1)
    pltpu.sync_copy(zn.at[pl.ds(sid * SL, SL)], acc.at[pl.ds(sid * SL, SL)])
    for hbm, v in ((z0, zv0), (z1, zv1), (z2, zv2),
                   (di0, dv0), (di1, dv1), (di2, dv2)):
        pltpu.sync_copy(hbm, v)
    plsc.subcore_barrier()
    zvs = (zv0, zv1, zv2)
    dvs = (dv0, dv1, dv2)
    eis = (e0, e1, e2)
    chunks = [(t, k) for t in range(3) for k in range(NCH)]
    tot = len(chunks)
    descs_s = [None] * tot
    for cnt in range(tot):
        t, k = chunks[cnt]
        b = cnt % 2
        base = wid * EWM + k * CW
        pltpu.sync_copy(eis[t].at[0, pl.ds(base, CW)], sidx)
        pltpu.sync_copy(eis[t].at[1, pl.ds(base, CW)], didx[b])
        zv = zvs[t]
        dv = dvs[t]
        vb = vals[b]
        db = didx[b]

        def body(i, _, zv=zv, dv=dv, vb=vb, db=db):
            idxs = sidx[pl.ds(i * 16, 16)]
            idxd = db[pl.ds(i * 16, 16)]
            v = plsc.load_gather(zv, [idxs]) * plsc.load_gather(dv, [idxd])
            vb[pl.ds(i * 16, 16)] = v
            return 0

        lax.fori_loop(0, CW // 16, body, 0, unroll=8)
        descs_s[cnt] = pltpu.async_copy(vb, acc.at[db], ssem[b], add=True)
        if cnt >= 1:
            descs_s[cnt - 1].wait()
    descs_s[tot - 1].wait()
    plsc.subcore_barrier()
    pltpu.sync_copy(acc.at[pl.ds(sid * SL, SL)],
                    out.at[cid, pl.ds(sid * SL, SL)])


def _tc1_body(degp_ref, x_ref, w_ref, dinv_ref, y0_ref, y1_ref, y2_ref,
              di0_ref, di1_ref, di2_ref, do0_ref, do1_ref, do2_ref):
    deg = jnp.maximum(degp_ref[0] + degp_ref[1], 1.0)
    dinv = lax.rsqrt(deg)
    dinv_ref[...] = dinv
    y = jnp.dot(x_ref[...], w_ref[...], preferred_element_type=_f32)
    for e, yr in enumerate((y0_ref, y1_ref, y2_ref)):
        yr[...] = y[:, 16 * e:16 * (e + 1)] * dinv[2 * e][:, None]
    for e, dr in enumerate((di0_ref, di1_ref, di2_ref)):
        dr[...] = dinv[2 * e + 1]
    for e, dr in enumerate((do0_ref, do1_ref, do2_ref)):
        dr[...] = dinv[2 * e]


def _tc2_body(aggp_ref, dr0_ref, dr1_ref, dr2_ref, b1t_ref, w2blk_ref,
              z0_ref, z1_ref, z2_ref):
    hp = jnp.zeros((NP8, 128), _f32)
    for e, dr in enumerate((dr0_ref, dr1_ref, dr2_ref)):
        hp = hp + (aggp_ref[0, e] + aggp_ref[1, e]) * dr[...]
    hp = jnp.maximum(hp + b1t_ref[0][None, :], 0.0)
    zp = jnp.dot(hp, w2blk_ref[...], preferred_element_type=_f32)
    for e, zr in enumerate((z0_ref, z1_ref, z2_ref)):
        zr[...] = zp[:, 8 * e:8 * (e + 1)]


def kernel(x, edge_index_rsr, edge_index_rtr, edge_index_rur,
           W1_rsr, b1_rsr, W1_rtr, b1_rtr, W1_rur, b1_rur,
           W2_rsr, b2_rsr, W2_rtr, b2_rtr, W2_rur, b2_rur):
    xpad = jnp.zeros((NPAD, IN_F), _f32).at[:N].set(x)
    w1 = jnp.concatenate([W1_rsr, W1_rtr, W1_rur], axis=1)
    w2 = jnp.concatenate([W2_rsr, W2_rtr, W2_rur], axis=1)
    b1t = jnp.tile(b1_rsr + b1_rtr + b1_rur, 8).reshape(1, 128)
    b2s = b2_rsr + b2_rtr + b2_rur
    eye8 = jnp.eye(8, dtype=_f32)
    w2blk = jnp.concatenate(
        [jnp.kron(eye8, w2[:, e:e + 1]) for e in range(3)], axis=1)
    zn = jnp.zeros((NPAD,), _f32)
    znk = jnp.zeros((NPAD, HID), _f32)
    ones = jnp.ones((EWM,), _f32)
    es = (edge_index_rsr, edge_index_rtr, edge_index_rur)

    degp = _sc_degrees(*es, zn, ones)

    (dinv, y0, y1, y2, di0, di1, di2, do0, do1, do2) = pl.pallas_call(
        _tc1_body,
        out_shape=(jax.ShapeDtypeStruct((6, NPAD), _f32),)
        + (jax.ShapeDtypeStruct((NPAD, HID), _f32),) * 3
        + (jax.ShapeDtypeStruct((NPAD,), _f32),) * 6,
    )(degp, xpad, w1)

    aggp = _sc_agg1(*es, y0, y1, y2, znk)

    drep = [jnp.repeat(d, 16).reshape(NP8, 128) for d in (di0, di1, di2)]
    zp0, zp1, zp2 = pl.pallas_call(
        _tc2_body,
        out_shape=(jax.ShapeDtypeStruct((NP8, 8), _f32),) * 3,
    )(aggp, *drep, b1t, w2blk)

    outp = _sc_agg2(*es, zp0.reshape(NPAD) * do0, zp1.reshape(NPAD) * do1,
                    zp2.reshape(NPAD) * do2, di0, di1, di2, zn)
    out = outp[0] + outp[1] + b2s[0]
    return out[:N]
